# baseline (speedup 1.0000x reference)
import jax
import jax.numpy as jnp
from jax.experimental import pallas as pl
from jax.experimental.pallas import tpu as pltpu

B = 1024
K = 1000
D = 512
MARGIN = 5.0

BB = 128
NB = B // BB


def _tc_body(x_ref, c_ref, lab_ref, out_ref, cch_ref, labc_ref):
    bb = pl.program_id(0)

    @pl.when(bb == 0)
    def _stage():
        c = c_ref[...]
        cch_ref[...] = 0.5 * jnp.sum(c * c, axis=1)[None, :]
        labc_ref[...] = lab_ref[...].reshape(B, 1)

    x = x_ref[...]
    prod = jax.lax.dot_general(
        x, c_ref[...], dimension_numbers=(((1,), (1,)), ((), ())),
        preferred_element_type=jnp.float32,
        precision=None)
    d2h = cch_ref[...] - prod
    lab = labc_ref[pl.ds(bb * BB, BB), :]
    own = jax.lax.broadcasted_iota(jnp.int32, (BB, K), 1) == lab
    neg = jnp.min(jnp.where(own, jnp.inf, d2h), axis=1, keepdims=True)
    pos = jnp.sum(jnp.where(own, d2h, 0.0), axis=1, keepdims=True)
    z = pos - neg + MARGIN
    partial = jnp.sum(jnp.log1p(jnp.exp(z))) / B

    @pl.when(bb == 0)
    def _first():
        out_ref[0, 0] = partial

    @pl.when(bb > 0)
    def _rest():
        out_ref[0, 0] += partial


@jax.jit
def kernel(x, labels, centers):
    loss = pl.pallas_call(
        _tc_body,
        grid=(NB,),
        in_specs=[
            pl.BlockSpec((BB, D), lambda b: (b, 0)),
            pl.BlockSpec((K, D), lambda b: (0, 0)),
            pl.BlockSpec((B,), lambda b: (0,)),
        ],
        out_specs=pl.BlockSpec(memory_space=pltpu.SMEM),
        out_shape=jax.ShapeDtypeStruct((1, 1), jnp.float32),
        scratch_shapes=[
            pltpu.VMEM((1, K), jnp.float32),
            pltpu.VMEM((B, 1), jnp.int32),
        ],
    )(x, centers, labels.astype(jnp.int32))
    return loss[0, 0]


# speedup vs baseline: 1.4430x; 1.4430x over previous
import jax
import jax.numpy as jnp
from jax.experimental import pallas as pl
from jax.experimental.pallas import tpu as pltpu

B = 1024
K = 1000
D = 512
MARGIN = 5.0

BB = 256
NB = B // BB


def _tc_body(x_ref, c_ref, lab_ref, out_ref, cch_ref, labc_ref):
    bb = pl.program_id(0)

    @pl.when(bb == 0)
    def _stage():
        c = c_ref[...]
        cch_ref[...] = 0.5 * jnp.sum(c * c, axis=1)[None, :]
        labc_ref[...] = lab_ref[...].reshape(B, 1)

    x = x_ref[...]
    prod = jax.lax.dot_general(
        x, c_ref[...], dimension_numbers=(((1,), (1,)), ((), ())),
        preferred_element_type=jnp.float32,
        precision=None)
    d2h = cch_ref[...] - prod
    lab = labc_ref[pl.ds(bb * BB, BB), :]
    own = jax.lax.broadcasted_iota(jnp.int32, (BB, K), 1) == lab
    neg = jnp.min(jnp.where(own, jnp.inf, d2h), axis=1, keepdims=True)
    pos = jnp.sum(jnp.where(own, d2h, 0.0), axis=1, keepdims=True)
    z = pos - neg + MARGIN
    partial = jnp.sum(jnp.log1p(jnp.exp(z))) / B

    @pl.when(bb == 0)
    def _first():
        out_ref[0, 0] = partial

    @pl.when(bb > 0)
    def _rest():
        out_ref[0, 0] += partial


@jax.jit
def kernel(x, labels, centers):
    loss = pl.pallas_call(
        _tc_body,
        grid=(NB,),
        in_specs=[
            pl.BlockSpec((BB, D), lambda b: (b, 0)),
            pl.BlockSpec((K, D), lambda b: (0, 0)),
            pl.BlockSpec((B,), lambda b: (0,)),
        ],
        out_specs=pl.BlockSpec(memory_space=pltpu.SMEM),
        out_shape=jax.ShapeDtypeStruct((1, 1), jnp.float32),
        scratch_shapes=[
            pltpu.VMEM((1, K), jnp.float32),
            pltpu.VMEM((B, 1), jnp.int32),
        ],
    )(x, centers, labels.astype(jnp.int32))
    return loss[0, 0]


# speedup vs baseline: 1.6724x; 1.1590x over previous
import jax
import jax.numpy as jnp
from jax.experimental import pallas as pl
from jax.experimental.pallas import tpu as pltpu

B = 1024
K = 1000
D = 512
MARGIN = 5.0

BB = 512
NB = B // BB


def _tc_body(x_ref, c_ref, lab_ref, out_ref, cch_ref, labc_ref):
    bb = pl.program_id(0)

    @pl.when(bb == 0)
    def _stage():
        c = c_ref[...]
        cch_ref[...] = 0.5 * jnp.sum(c * c, axis=1)[None, :]
        labc_ref[...] = lab_ref[...].reshape(B, 1)

    x = x_ref[...]
    prod = jax.lax.dot_general(
        x, c_ref[...], dimension_numbers=(((1,), (1,)), ((), ())),
        preferred_element_type=jnp.float32,
        precision=None)
    d2h = cch_ref[...] - prod
    lab = labc_ref[pl.ds(bb * BB, BB), :]
    own = jax.lax.broadcasted_iota(jnp.int32, (BB, K), 1) == lab
    neg = jnp.min(jnp.where(own, jnp.inf, d2h), axis=1, keepdims=True)
    pos = jnp.sum(jnp.where(own, d2h, 0.0), axis=1, keepdims=True)
    z = pos - neg + MARGIN
    partial = jnp.sum(jnp.log1p(jnp.exp(z))) / B

    @pl.when(bb == 0)
    def _first():
        out_ref[0, 0] = partial

    @pl.when(bb > 0)
    def _rest():
        out_ref[0, 0] += partial


@jax.jit
def kernel(x, labels, centers):
    loss = pl.pallas_call(
        _tc_body,
        grid=(NB,),
        in_specs=[
            pl.BlockSpec((BB, D), lambda b: (b, 0)),
            pl.BlockSpec((K, D), lambda b: (0, 0)),
            pl.BlockSpec((B,), lambda b: (0,)),
        ],
        out_specs=pl.BlockSpec(memory_space=pltpu.SMEM),
        out_shape=jax.ShapeDtypeStruct((1, 1), jnp.float32),
        scratch_shapes=[
            pltpu.VMEM((1, K), jnp.float32),
            pltpu.VMEM((B, 1), jnp.int32),
        ],
    )(x, centers, labels.astype(jnp.int32))
    return loss[0, 0]


# speedup vs baseline: 1.9453x; 1.1632x over previous
import jax
import jax.numpy as jnp
from jax.experimental import pallas as pl
from jax.experimental.pallas import tpu as pltpu

B = 1024
K = 1000
D = 512
MARGIN = 5.0

BB = 1024
NB = B // BB


def _tc_body(x_ref, c_ref, lab_ref, out_ref, cch_ref, labc_ref):
    bb = pl.program_id(0)

    @pl.when(bb == 0)
    def _stage():
        c = c_ref[...]
        cch_ref[...] = 0.5 * jnp.sum(c * c, axis=1)[None, :]
        labc_ref[...] = lab_ref[...].reshape(B, 1)

    x = x_ref[...]
    prod = jax.lax.dot_general(
        x, c_ref[...], dimension_numbers=(((1,), (1,)), ((), ())),
        preferred_element_type=jnp.float32,
        precision=None)
    d2h = cch_ref[...] - prod
    lab = labc_ref[pl.ds(bb * BB, BB), :]
    own = jax.lax.broadcasted_iota(jnp.int32, (BB, K), 1) == lab
    neg = jnp.min(jnp.where(own, jnp.inf, d2h), axis=1, keepdims=True)
    pos = jnp.sum(jnp.where(own, d2h, 0.0), axis=1, keepdims=True)
    z = pos - neg + MARGIN
    partial = jnp.sum(jnp.log1p(jnp.exp(z))) / B

    @pl.when(bb == 0)
    def _first():
        out_ref[0, 0] = partial

    @pl.when(bb > 0)
    def _rest():
        out_ref[0, 0] += partial


@jax.jit
def kernel(x, labels, centers):
    loss = pl.pallas_call(
        _tc_body,
        grid=(NB,),
        in_specs=[
            pl.BlockSpec((BB, D), lambda b: (b, 0)),
            pl.BlockSpec((K, D), lambda b: (0, 0)),
            pl.BlockSpec((B,), lambda b: (0,)),
        ],
        out_specs=pl.BlockSpec(memory_space=pltpu.SMEM),
        out_shape=jax.ShapeDtypeStruct((1, 1), jnp.float32),
        scratch_shapes=[
            pltpu.VMEM((1, K), jnp.float32),
            pltpu.VMEM((B, 1), jnp.int32),
        ],
    )(x, centers, labels.astype(jnp.int32))
    return loss[0, 0]
